# baseline (speedup 1.0000x reference)
import functools

import jax
import jax.numpy as jnp
from jax import lax
from jax.experimental import pallas as pl
from jax.experimental.pallas import tpu as pltpu
from jax.experimental.pallas import tpu_sc as plsc

B, T, C, H, HS = 4, 10000, 128, 8, 16
E = 320000
N = B * T

_K = 80
_NCH = 250
_G = 10
_NB = _NCH // _G
_DW = 16
_NSUB = 16
_RPS_A = 632
_RPS_B = T - 15 * _RPS_A
_ZR = 64


def _tc_pre(x2d, ln1_g, ln1_b, W_gat, attA):
    R = 800

    def body(x_ref, g_ref, b_ref, W_ref, A_ref, xw_ref, as_ref, ad_ref):
        x = x_ref[...]
        m = jnp.mean(x, axis=1, keepdims=True)
        xc = x - m
        v = jnp.mean(xc * xc, axis=1, keepdims=True)
        h = xc * lax.rsqrt(v + 1e-5) * g_ref[...] + b_ref[...]
        xw = jnp.dot(h, W_ref[...], preferred_element_type=jnp.float32)
        xw_ref[...] = xw
        asd = jnp.dot(xw, A_ref[...], preferred_element_type=jnp.float32)
        as_ref[...] = asd[:, :16]
        ad_ref[...] = asd[:, 16:]

    return pl.pallas_call(
        body,
        grid=(N // R,),
        in_specs=[
            pl.BlockSpec((R, 128), lambda i: (i, 0)),
            pl.BlockSpec((1, 128), lambda i: (0, 0)),
            pl.BlockSpec((1, 128), lambda i: (0, 0)),
            pl.BlockSpec((128, 128), lambda i: (0, 0)),
            pl.BlockSpec((128, 32), lambda i: (0, 0)),
        ],
        out_specs=[
            pl.BlockSpec((R, 128), lambda i: (i, 0)),
            pl.BlockSpec((R, 16), lambda i: (i, 0)),
            pl.BlockSpec((R, 16), lambda i: (i, 0)),
        ],
        out_shape=[
            jax.ShapeDtypeStruct((N, 128), jnp.float32),
            jax.ShapeDtypeStruct((N, 16), jnp.float32),
            jax.ShapeDtypeStruct((N, 16), jnp.float32),
        ],
    )(x2d, ln1_g.reshape(1, 128), ln1_b.reshape(1, 128), W_gat, attA)


def _sc_edge(src3, dst3, as_tab, ad_tab, xw):
    mesh = plsc.VectorSubcoreMesh(core_axis_name="c", subcore_axis_name="s")

    @functools.partial(
        pl.kernel,
        out_type=[
            jax.ShapeDtypeStruct((N, 128), jnp.float32),
            jax.ShapeDtypeStruct((N, _DW), jnp.float32),
        ],
        mesh=mesh,
        compiler_params=pltpu.CompilerParams(use_tc_tiling_on_sc=False),
        scratch_types=(
            [pltpu.VMEM((_K,), jnp.int32)] * 2
            + [pltpu.VMEM((_K,), jnp.int32)] * 2
            + [pltpu.VMEM((_K,), jnp.int32)] * 2
            + [pltpu.VMEM((_K,), jnp.int32)] * 5
            + [pltpu.VMEM((_K, 16), jnp.float32)] * 2
            + [pltpu.VMEM((_K, 16), jnp.float32)] * 2
            + [pltpu.VMEM((_K, 16), jnp.float32)] * 3
            + [pltpu.VMEM((_K, 128), jnp.float32)] * 3
            + [
                pltpu.VMEM_SHARED((T, 128), jnp.float32),
                pltpu.VMEM_SHARED((T, _DW), jnp.float32),
            ]
            + [pltpu.SemaphoreType.DMA] * 14
        ),
    )
    def k(src_hbm, dst_hbm, as_hbm, ad_hbm, xw_hbm, numer_hbm, denom_hbm,
          *refs):
        srcc = refs[0:2]
        srcg = refs[2:4]
        dstg = refs[4:6]
        dstl = refs[6:11]
        av = refs[11:13]
        bv = refs[13:15]
        wv = refs[15:18]
        rows = refs[18:21]
        nacc, dacc = refs[21], refs[22]
        sga = refs[23:25]
        sgb = refs[25:27]
        sgr = refs[27:29]
        ssn = refs[29:32]
        ssd = refs[32:35]
        sidx = refs[35:37]

        c = lax.axis_index("c")
        s = lax.axis_index("s")

        def start_idx(i, p2, p5):
            da = pltpu.async_copy(src_hbm.at[s, i], srcc[p2], sidx[p2])
            db = pltpu.async_copy(dst_hbm.at[s, i], dstl[p5], sidx[p2])
            return (da, db)

        def globalize(p2, p5, off):
            for j in range(_K // 16):
                sl = pl.ds(j * 16, 16)
                srcg[p2][sl] = srcc[p2][sl] + off
                dstg[p2][sl] = dstl[p5][sl] + off

        def start_gathers(p3, p2):
            ga = pltpu.async_copy(as_hbm.at[srcg[p2]], av[p2], sga[p2])
            gb = pltpu.async_copy(ad_hbm.at[dstg[p2]], bv[p2], sgb[p2])
            gr = pltpu.async_copy(xw_hbm.at[srcg[p2]], rows[p3], sgr[p2])
            return (ga, gb, gr)

        def compute(p3, p2):
            def edge(e, carry):
                vsum = av[p2][e] + bv[p2][e]
                w16 = jnp.exp(jnp.where(vsum >= 0.0, vsum, vsum * 0.2))
                wv[p3][e] = w16
                for h in range(8):
                    sl = pl.ds(h * 16, 16)
                    rows[p3][e, sl] = rows[p3][e, sl] * w16[h]
                return carry

            lax.fori_loop(0, _K, edge, None, unroll=2)

        def zero_slice(base, nrows):
            for q in range(nrows // _K):
                pltpu.sync_copy(rows[0], nacc.at[pl.ds(base + q * _K, _K)])
                pltpu.sync_copy(wv[0], dacc.at[pl.ds(base + q * _K, _K)])
            rem = nrows % _K
            if rem:
                rb = base + (nrows // _K) * _K
                pltpu.sync_copy(rows[0].at[pl.ds(0, rem)],
                                nacc.at[pl.ds(rb, rem)])
                pltpu.sync_copy(wv[0].at[pl.ds(0, rem)],
                                dacc.at[pl.ds(rb, rem)])

        for bi in range(2):
            off = pl.multiple_of((c * 2 + bi) * T, 8)

            def zstore(e, carry):
                for h in range(8):
                    rows[0][e, pl.ds(h * 16, 16)] = jnp.zeros((16,),
                                                              jnp.float32)
                wv[0][e] = jnp.zeros((16,), jnp.float32)
                return carry

            lax.fori_loop(0, _K, zstore, None)

            @pl.when(s < 15)
            def _():
                zero_slice(s * _RPS_A, _RPS_A)

            @pl.when(s == 15)
            def _():
                zero_slice(15 * _RPS_A, _RPS_B)

            plsc.subcore_barrier()

            def body(g, carry):
                c0 = g * _G
                dg = [None] * _G
                dsc = [None] * _G
                di = [None] * _G
                di[0] = start_idx(c0, 0, 0)
                di[1] = start_idx(c0 + 1, 1, 1)
                for d in di[0]:
                    d.wait()
                globalize(0, 0, off)
                dg[0] = start_gathers(0, 0)
                for t in range(_G):
                    p3, p2 = t % 3, t % 2
                    if t < _G - 1:
                        if t >= 2:
                            for d in dsc[t - 2]:
                                d.wait()
                        for d in di[t + 1]:
                            d.wait()
                        globalize((t + 1) % 2, (t + 1) % 5, off)
                        dg[t + 1] = start_gathers((t + 1) % 3, (t + 1) % 2)
                    if t < _G - 2:
                        di[t + 2] = start_idx(c0 + t + 2, (t + 2) % 2,
                                              (t + 2) % 5)
                    for d in dg[t]:
                        d.wait()
                    compute(p3, p2)
                    dsc[t] = (pltpu.async_copy(rows[p3], nacc.at[dstl[t % 5]],
                                               ssn[p3], add=True),
                              pltpu.async_copy(wv[p3], dacc.at[dstl[t % 5]],
                                               ssd[p3], add=True))
                for d in dsc[_G - 2] + dsc[_G - 1]:
                    d.wait()
                return carry

            lax.fori_loop(0, _NB, body, None)
            plsc.subcore_barrier()

            @pl.when(s < 15)
            def _():
                base = s * _RPS_A
                pltpu.sync_copy(nacc.at[pl.ds(base, _RPS_A)],
                                numer_hbm.at[pl.ds(off + base, _RPS_A)])
                pltpu.sync_copy(dacc.at[pl.ds(base, _RPS_A)],
                                denom_hbm.at[pl.ds(off + base, _RPS_A)])

            @pl.when(s == 15)
            def _():
                base = 15 * _RPS_A
                pltpu.sync_copy(nacc.at[pl.ds(base, _RPS_B)],
                                numer_hbm.at[pl.ds(off + base, _RPS_B)])
                pltpu.sync_copy(dacc.at[pl.ds(base, _RPS_B)],
                                denom_hbm.at[pl.ds(off + base, _RPS_B)])

    return k(src3, dst3, as_tab, ad_tab, xw)


def _tc_post(x2d, numer, denom, Erep, b_gat, ln2_g, ln2_b, W1, b1, W2, b2):
    R = 800

    def body(x_ref, n_ref, d_ref, E_ref, bg_ref, g_ref, b_ref,
             W1_ref, b1_ref, W2_ref, b2_ref, o_ref):
        r = 1.0 / (d_ref[...] + 1e-16)
        rex = jnp.dot(r, E_ref[...], preferred_element_type=jnp.float32)
        gat = n_ref[...] * rex + bg_ref[...]
        x2 = x_ref[...] + gat
        m = jnp.mean(x2, axis=1, keepdims=True)
        xc = x2 - m
        v = jnp.mean(xc * xc, axis=1, keepdims=True)
        h2 = xc * lax.rsqrt(v + 1e-5) * g_ref[...] + b_ref[...]
        f1 = jnp.maximum(
            jnp.dot(h2, W1_ref[...], preferred_element_type=jnp.float32)
            + b1_ref[...], 0.0)
        ff = jnp.dot(f1, W2_ref[...], preferred_element_type=jnp.float32) \
            + b2_ref[...]
        o_ref[...] = x2 + ff

    return pl.pallas_call(
        body,
        grid=(N // R,),
        in_specs=[
            pl.BlockSpec((R, 128), lambda i: (i, 0)),
            pl.BlockSpec((R, 128), lambda i: (i, 0)),
            pl.BlockSpec((R, _DW), lambda i: (i, 0)),
            pl.BlockSpec((_DW, 128), lambda i: (0, 0)),
            pl.BlockSpec((1, 128), lambda i: (0, 0)),
            pl.BlockSpec((1, 128), lambda i: (0, 0)),
            pl.BlockSpec((1, 128), lambda i: (0, 0)),
            pl.BlockSpec((128, 512), lambda i: (0, 0)),
            pl.BlockSpec((1, 512), lambda i: (0, 0)),
            pl.BlockSpec((512, 128), lambda i: (0, 0)),
            pl.BlockSpec((1, 128), lambda i: (0, 0)),
        ],
        out_specs=pl.BlockSpec((R, 128), lambda i: (i, 0)),
        out_shape=jax.ShapeDtypeStruct((N, 128), jnp.float32),
    )(x2d, numer, denom, Erep, b_gat.reshape(1, 128), ln2_g.reshape(1, 128),
      ln2_b.reshape(1, 128), W1, b1.reshape(1, 512), W2, b2.reshape(1, 128))


def kernel(x, edge_index, W_gat, att_src, att_dst, b_gat,
           ln1_g, ln1_b, ln2_g, ln2_b, W1, b1, W2, b2):
    x2d = x.reshape(N, C)

    eye8 = jnp.eye(8, dtype=jnp.float32)
    Asrc = (att_src[:, :, None] * eye8[:, None, :]).reshape(C, 8)
    Adst = (att_dst[:, :, None] * eye8[:, None, :]).reshape(C, 8)
    attA = jnp.concatenate([Asrc, Asrc, Adst, Adst], axis=1)

    Erep = (eye8[:, :, None] * jnp.ones((1, 1, 16), jnp.float32)).reshape(8, C)
    if _DW > 8:
        Erep = jnp.concatenate(
            [Erep, jnp.zeros((_DW - 8, C), jnp.float32)], axis=0)

    src3 = edge_index[0].reshape(_NSUB, _NCH, _K)
    dst3 = edge_index[1].reshape(_NSUB, _NCH, _K)

    xw, as_tab, ad_tab = _tc_pre(x2d, ln1_g, ln1_b, W_gat, attA)
    numer, denom = _sc_edge(src3, dst3, as_tab, ad_tab, xw)
    out2d = _tc_post(x2d, numer, denom, Erep, b_gat, ln2_g, ln2_b,
                     W1, b1, W2, b2)
    return out2d.reshape(B, T, C)


# speedup vs baseline: 1.0833x; 1.0833x over previous
import functools

import jax
import jax.numpy as jnp
from jax import lax
from jax.experimental import pallas as pl
from jax.experimental.pallas import tpu as pltpu
from jax.experimental.pallas import tpu_sc as plsc

B, T, C, H, HS = 4, 10000, 128, 8, 16
E = 320000
N = B * T

_K = 80
_NCH = 250
_G = 25
_NB = _NCH // _G
_DW = 16
_NSUB = 16
_RPS_A = 632
_RPS_B = T - 15 * _RPS_A
_ZR = 64


def _tc_pre(x2d, ln1_g, ln1_b, W_gat, attA):
    R = 800

    def body(x_ref, g_ref, b_ref, W_ref, A_ref, xw_ref, as_ref, ad_ref):
        x = x_ref[...]
        m = jnp.mean(x, axis=1, keepdims=True)
        xc = x - m
        v = jnp.mean(xc * xc, axis=1, keepdims=True)
        h = xc * lax.rsqrt(v + 1e-5) * g_ref[...] + b_ref[...]
        xw = jnp.dot(h, W_ref[...], preferred_element_type=jnp.float32)
        xw_ref[...] = xw
        asd = jnp.dot(xw, A_ref[...], preferred_element_type=jnp.float32)
        as_ref[...] = asd[:, :16]
        ad_ref[...] = asd[:, 16:]

    return pl.pallas_call(
        body,
        grid=(N // R,),
        in_specs=[
            pl.BlockSpec((R, 128), lambda i: (i, 0)),
            pl.BlockSpec((1, 128), lambda i: (0, 0)),
            pl.BlockSpec((1, 128), lambda i: (0, 0)),
            pl.BlockSpec((128, 128), lambda i: (0, 0)),
            pl.BlockSpec((128, 32), lambda i: (0, 0)),
        ],
        out_specs=[
            pl.BlockSpec((R, 128), lambda i: (i, 0)),
            pl.BlockSpec((R, 16), lambda i: (i, 0)),
            pl.BlockSpec((R, 16), lambda i: (i, 0)),
        ],
        out_shape=[
            jax.ShapeDtypeStruct((N, 128), jnp.float32),
            jax.ShapeDtypeStruct((N, 16), jnp.float32),
            jax.ShapeDtypeStruct((N, 16), jnp.float32),
        ],
    )(x2d, ln1_g.reshape(1, 128), ln1_b.reshape(1, 128), W_gat, attA)


def _sc_edge(src3, dst3, as_tab, ad_tab, xw):
    mesh = plsc.VectorSubcoreMesh(core_axis_name="c", subcore_axis_name="s")

    @functools.partial(
        pl.kernel,
        out_type=[
            jax.ShapeDtypeStruct((N, 128), jnp.float32),
            jax.ShapeDtypeStruct((N, _DW), jnp.float32),
        ],
        mesh=mesh,
        compiler_params=pltpu.CompilerParams(use_tc_tiling_on_sc=False),
        scratch_types=(
            [pltpu.VMEM((_K,), jnp.int32)] * 3
            + [pltpu.VMEM((_K,), jnp.int32)] * 5
            + [pltpu.VMEM((_K, 16), jnp.float32)] * 2
            + [pltpu.VMEM((_K, 16), jnp.float32)] * 2
            + [pltpu.VMEM((_K, 16), jnp.float32)] * 3
            + [pltpu.VMEM((_K, 128), jnp.float32)] * 3
            + [
                pltpu.VMEM_SHARED((T, 128), jnp.float32),
                pltpu.VMEM_SHARED((T, _DW), jnp.float32),
            ]
            + [pltpu.SemaphoreType.DMA] * 14
        ),
    )
    def k(src_hbm, dst_hbm, as_hbm, ad_hbm, xw_hbm, numer_hbm, denom_hbm,
          *refs):
        srcc = refs[0:3]
        dstl = refs[3:8]
        av = refs[8:10]
        bv = refs[10:12]
        wv = refs[12:15]
        rows = refs[15:18]
        nacc, dacc = refs[18], refs[19]
        sga = refs[20:22]
        sgb = refs[22:24]
        sgr = refs[24:26]
        ssn = refs[26:29]
        ssd = refs[29:32]
        sidx = refs[32:34]

        c = lax.axis_index("c")
        s = lax.axis_index("s")

        def start_idx(i, p3c, p5):
            da = pltpu.async_copy(src_hbm.at[s, i], srcc[p3c], sidx[p3c % 2])
            db = pltpu.async_copy(dst_hbm.at[s, i], dstl[p5], sidx[p3c % 2])
            return (da, db)

        def start_gathers(asb, adb, xwb, p3, p2, p3c, p5):
            ga = pltpu.async_copy(asb.at[srcc[p3c]], av[p2], sga[p2])
            gb = pltpu.async_copy(adb.at[dstl[p5]], bv[p2], sgb[p2])
            gr = pltpu.async_copy(xwb.at[srcc[p3c]], rows[p3], sgr[p2])
            return (ga, gb, gr)

        def compute(p3, p2):
            def edge(e, carry):
                vsum = av[p2][e] + bv[p2][e]
                w16 = jnp.exp(jnp.where(vsum >= 0.0, vsum, vsum * 0.2))
                wv[p3][e] = w16
                for h in range(8):
                    sl = pl.ds(h * 16, 16)
                    rows[p3][e, sl] = rows[p3][e, sl] * w16[h]
                return carry

            lax.fori_loop(0, _K, edge, None)

        def zero_slice(base, nrows):
            for q in range(nrows // _K):
                pltpu.sync_copy(rows[0], nacc.at[pl.ds(base + q * _K, _K)])
                pltpu.sync_copy(wv[0], dacc.at[pl.ds(base + q * _K, _K)])
            rem = nrows % _K
            if rem:
                rb = base + (nrows // _K) * _K
                pltpu.sync_copy(rows[0].at[pl.ds(0, rem)],
                                nacc.at[pl.ds(rb, rem)])
                pltpu.sync_copy(wv[0].at[pl.ds(0, rem)],
                                dacc.at[pl.ds(rb, rem)])

        for bi in range(2):
            off = pl.multiple_of((c * 2 + bi) * T, 8)

            def zstore(e, carry):
                for h in range(8):
                    rows[0][e, pl.ds(h * 16, 16)] = jnp.zeros((16,),
                                                              jnp.float32)
                wv[0][e] = jnp.zeros((16,), jnp.float32)
                return carry

            lax.fori_loop(0, _K, zstore, None)

            @pl.when(s < 15)
            def _():
                zero_slice(s * _RPS_A, _RPS_A)

            @pl.when(s == 15)
            def _():
                zero_slice(15 * _RPS_A, _RPS_B)

            plsc.subcore_barrier()

            b = c * 2 + bi
            asb = as_hbm.at[b]
            adb = ad_hbm.at[b]
            xwb = xw_hbm.at[b]

            def body(g, carry):
                c0 = g * _G
                dg = [None] * _G
                dsc = [None] * _G
                di = [None] * _G
                di[0] = start_idx(c0, 0, 0)
                di[1] = start_idx(c0 + 1, 1, 1)
                for d in di[0]:
                    d.wait()
                dg[0] = start_gathers(asb, adb, xwb, 0, 0, 0, 0)
                for t in range(_G):
                    p3, p2 = t % 3, t % 2
                    if t < _G - 1:
                        if t >= 2:
                            for d in dsc[t - 2]:
                                d.wait()
                        for d in di[t + 1]:
                            d.wait()
                        dg[t + 1] = start_gathers(asb, adb, xwb, (t + 1) % 3,
                                                  (t + 1) % 2, (t + 1) % 3,
                                                  (t + 1) % 5)
                    if t < _G - 2:
                        di[t + 2] = start_idx(c0 + t + 2, (t + 2) % 3,
                                              (t + 2) % 5)
                    for d in dg[t]:
                        d.wait()
                    compute(p3, p2)
                    dsc[t] = (pltpu.async_copy(rows[p3], nacc.at[dstl[t % 5]],
                                               ssn[p3], add=True),
                              pltpu.async_copy(wv[p3], dacc.at[dstl[t % 5]],
                                               ssd[p3], add=True))
                for d in dsc[_G - 2] + dsc[_G - 1]:
                    d.wait()
                return carry

            lax.fori_loop(0, _NB, body, None)
            plsc.subcore_barrier()

            @pl.when(s < 15)
            def _():
                base = s * _RPS_A
                pltpu.sync_copy(nacc.at[pl.ds(base, _RPS_A)],
                                numer_hbm.at[pl.ds(off + base, _RPS_A)])
                pltpu.sync_copy(dacc.at[pl.ds(base, _RPS_A)],
                                denom_hbm.at[pl.ds(off + base, _RPS_A)])

            @pl.when(s == 15)
            def _():
                base = 15 * _RPS_A
                pltpu.sync_copy(nacc.at[pl.ds(base, _RPS_B)],
                                numer_hbm.at[pl.ds(off + base, _RPS_B)])
                pltpu.sync_copy(dacc.at[pl.ds(base, _RPS_B)],
                                denom_hbm.at[pl.ds(off + base, _RPS_B)])

    return k(src3, dst3, as_tab, ad_tab, xw)


def _tc_post(x2d, numer, denom, Erep, b_gat, ln2_g, ln2_b, W1, b1, W2, b2):
    R = 800

    def body(x_ref, n_ref, d_ref, E_ref, bg_ref, g_ref, b_ref,
             W1_ref, b1_ref, W2_ref, b2_ref, o_ref):
        r = 1.0 / (d_ref[...] + 1e-16)
        rex = jnp.dot(r, E_ref[...], preferred_element_type=jnp.float32)
        gat = n_ref[...] * rex + bg_ref[...]
        x2 = x_ref[...] + gat
        m = jnp.mean(x2, axis=1, keepdims=True)
        xc = x2 - m
        v = jnp.mean(xc * xc, axis=1, keepdims=True)
        h2 = xc * lax.rsqrt(v + 1e-5) * g_ref[...] + b_ref[...]
        f1 = jnp.maximum(
            jnp.dot(h2, W1_ref[...], preferred_element_type=jnp.float32)
            + b1_ref[...], 0.0)
        ff = jnp.dot(f1, W2_ref[...], preferred_element_type=jnp.float32) \
            + b2_ref[...]
        o_ref[...] = x2 + ff

    return pl.pallas_call(
        body,
        grid=(N // R,),
        in_specs=[
            pl.BlockSpec((R, 128), lambda i: (i, 0)),
            pl.BlockSpec((R, 128), lambda i: (i, 0)),
            pl.BlockSpec((R, _DW), lambda i: (i, 0)),
            pl.BlockSpec((_DW, 128), lambda i: (0, 0)),
            pl.BlockSpec((1, 128), lambda i: (0, 0)),
            pl.BlockSpec((1, 128), lambda i: (0, 0)),
            pl.BlockSpec((1, 128), lambda i: (0, 0)),
            pl.BlockSpec((128, 512), lambda i: (0, 0)),
            pl.BlockSpec((1, 512), lambda i: (0, 0)),
            pl.BlockSpec((512, 128), lambda i: (0, 0)),
            pl.BlockSpec((1, 128), lambda i: (0, 0)),
        ],
        out_specs=pl.BlockSpec((R, 128), lambda i: (i, 0)),
        out_shape=jax.ShapeDtypeStruct((N, 128), jnp.float32),
    )(x2d, numer, denom, Erep, b_gat.reshape(1, 128), ln2_g.reshape(1, 128),
      ln2_b.reshape(1, 128), W1, b1.reshape(1, 512), W2, b2.reshape(1, 128))


def kernel(x, edge_index, W_gat, att_src, att_dst, b_gat,
           ln1_g, ln1_b, ln2_g, ln2_b, W1, b1, W2, b2):
    x2d = x.reshape(N, C)

    eye8 = jnp.eye(8, dtype=jnp.float32)
    Asrc = (att_src[:, :, None] * eye8[:, None, :]).reshape(C, 8)
    Adst = (att_dst[:, :, None] * eye8[:, None, :]).reshape(C, 8)
    attA = jnp.concatenate([Asrc, Asrc, Adst, Adst], axis=1)

    Erep = (eye8[:, :, None] * jnp.ones((1, 1, 16), jnp.float32)).reshape(8, C)
    if _DW > 8:
        Erep = jnp.concatenate(
            [Erep, jnp.zeros((_DW - 8, C), jnp.float32)], axis=0)

    src3 = edge_index[0].reshape(_NSUB, _NCH, _K)
    dst3 = edge_index[1].reshape(_NSUB, _NCH, _K)

    xw, as_tab, ad_tab = _tc_pre(x2d, ln1_g, ln1_b, W_gat, attA)
    numer, denom = _sc_edge(src3, dst3, as_tab.reshape(B, T, 16),
                            ad_tab.reshape(B, T, 16), xw.reshape(B, T, C))
    out2d = _tc_post(x2d, numer, denom, Erep, b_gat, ln2_g, ln2_b,
                     W1, b1, W2, b2)
    return out2d.reshape(B, T, C)
